# Initial kernel scaffold; baseline (speedup 1.0000x reference)
import jax
import jax.numpy as jnp
from jax.experimental import pallas as pl


def kernel(x, edge_index, W1, a_src1, a_dst1, b1, W2, a_src2, a_dst2, b2):
    raise NotImplementedError("write your pallas kernel here")


# speedup vs baseline: 49.5157x; 49.5157x over previous
import functools

import jax
import jax.numpy as jnp
from jax import lax
from jax.experimental import pallas as pl
from jax.experimental.pallas import tpu as pltpu
from jax.experimental.pallas import tpu_sc as plsc

N = 10000
E = 320000
D = 128
NP = 10240
NC, NS, L = 2, 16, 16
NW = NC * NS
EPW = E // NW
CH = 64
NCH = 160
EPW_PAD = NCH * CH
MBLK = 1024
GB = NP // MBLK
RPT = NP // NS


def _splat(v16, j):
    idx = jnp.full((16, 1), j, jnp.int32)
    dnums = lax.GatherDimensionNumbers(
        offset_dims=(), collapsed_slice_dims=(0,), start_index_map=(0,))
    return lax.gather(v16, idx, dnums, (1,),
                      mode=lax.GatherScatterMode.PROMISE_IN_BOUNDS)


@functools.partial(
    pl.kernel,
    out_type=(
        jax.ShapeDtypeStruct((NC, NP, D), jnp.float32),
        jax.ShapeDtypeStruct((NC, NP), jnp.float32),
    ),
    mesh=plsc.VectorSubcoreMesh(core_axis_name="c", subcore_axis_name="s"),
    compiler_params=pltpu.CompilerParams(needs_layout_passes=False),
    scratch_types=[
        pltpu.VMEM((NP,), jnp.float32),
        pltpu.VMEM((NP,), jnp.float32),
        pltpu.VMEM((16,), jnp.float32),
        pltpu.VMEM((CH,), jnp.int32),
        pltpu.VMEM((CH,), jnp.int32),
        pltpu.VMEM((CH,), jnp.int32),
        pltpu.VMEM((CH,), jnp.int32),
        pltpu.VMEM((CH,), jnp.int32),
        pltpu.VMEM((CH,), jnp.int32),
        pltpu.VMEM((CH,), jnp.int32),
        pltpu.VMEM((CH,), jnp.int32),
        pltpu.VMEM((CH, D), jnp.float32),
        pltpu.VMEM((CH, D), jnp.float32),
        pltpu.VMEM((CH,), jnp.float32),
        pltpu.VMEM((CH,), jnp.float32),
        pltpu.VMEM_SHARED((NP, D), jnp.float32),
        pltpu.VMEM_SHARED((NP,), jnp.float32),
        pltpu.SemaphoreType.DMA,
        pltpu.SemaphoreType.DMA,
        pltpu.SemaphoreType.DMA,
        pltpu.SemaphoreType.DMA,
        pltpu.SemaphoreType.DMA,
        pltpu.SemaphoreType.DMA,
        pltpu.SemaphoreType.DMA,
        pltpu.SemaphoreType.DMA,
        pltpu.SemaphoreType.DMA,
        pltpu.SemaphoreType.DMA,
        pltpu.SemaphoreType.DMA,
        pltpu.SemaphoreType.DMA,
    ],
)
def _sc_edge(h_hbm, as_hbm, ad_hbm, c_hbm, src_hbm, dst_hbm,
             outp_hbm, denp_hbm,
             as_v, ad_v, c_v,
             sb0, sb1, sb2, sb3, db0, db1, db2, db3,
             rb0, rb1, wb0, wb1, acc_sp, den_sp,
             gs0, gs1, ss0, ss1,
             is0, is1, is2, is3, id0, id1, id2, id3):
    cid = lax.axis_index("c")
    sid = lax.axis_index("s")
    wid = cid * NS + sid
    srcb = (sb0, sb1, sb2, sb3)
    dstb = (db0, db1, db2, db3)
    rowb = (rb0, rb1)
    wbuf = (wb0, wb1)
    gsem = (gs0, gs1)
    ssem = (ss0, ss1)
    isem_s = (is0, is1, is2, is3)
    isem_d = (id0, id1, id2, id3)

    pltpu.sync_copy(as_hbm, as_v)
    pltpu.sync_copy(ad_hbm, ad_v)
    pltpu.sync_copy(c_hbm, c_v)

    def _zr(i, _):
        rb0[i // 8, pl.ds((i % 8) * 16, 16)] = jnp.zeros((16,), jnp.float32)
        return 0
    lax.fori_loop(0, CH * 8, _zr, 0)

    def _zw(i, _):
        wb0[pl.ds(i * 16, 16)] = jnp.zeros((16,), jnp.float32)
        return 0
    lax.fori_loop(0, CH // 16, _zw, 0)

    for t in range(RPT // CH):
        pltpu.sync_copy(rb0, acc_sp.at[pl.ds(sid * RPT + t * CH, CH)])
        pltpu.sync_copy(wb0, den_sp.at[pl.ds(sid * RPT + t * CH, CH)])
    rem = RPT - (RPT // CH) * CH
    if rem:
        base = sid * RPT + (RPT // CH) * CH
        pltpu.sync_copy(rb0.at[pl.ds(0, rem)], acc_sp.at[pl.ds(base, rem)])
        pltpu.sync_copy(wb0.at[pl.ds(0, rem)], den_sp.at[pl.ds(base, rem)])
    plsc.subcore_barrier()

    c_off = c_v[...]

    for t in range(3):
        pltpu.async_copy(src_hbm.at[wid, t], srcb[t], isem_s[t])
        pltpu.async_copy(dst_hbm.at[wid, t], dstb[t], isem_d[t])
    pltpu.make_async_copy(src_hbm.at[wid, 0], srcb[0], isem_s[0]).wait()
    pltpu.async_copy(h_hbm.at[srcb[0]], rb0, gsem[0])

    def _quad(i, _):
        for k in range(4):
            c = i * 4 + k
            b = k % 2
            pltpu.make_async_copy(
                dst_hbm.at[wid, 0], dstb[k], isem_d[k]).wait()

            def _wgrp(g, _):
                s = srcb[k][pl.ds(g * 16, 16)]
                d = dstb[k][pl.ds(g * 16, 16)]
                e = plsc.load_gather(as_v, [s]) + plsc.load_gather(ad_v, [d])
                e = jnp.where(e >= 0.0, e, 0.2 * e)
                w = jnp.exp(e - c_off)
                pos = c * CH + g * 16 + lax.iota(jnp.int32, 16)
                w = jnp.where(pos < EPW, w, 0.0)
                wbuf[b][pl.ds(g * 16, 16)] = w
                return 0
            lax.fori_loop(0, CH // 16, _wgrp, 0)

            pltpu.sync_copy(wbuf[b], den_sp.at[dstb[k]], add=True)

            @pl.when(c >= 1)
            def _():
                pltpu.make_async_copy(
                    rowb[1 - b], acc_sp.at[dstb[(k - 1) % 4]],
                    ssem[1 - b]).wait()

            @pl.when(c + 1 < NCH)
            def _():
                pltpu.make_async_copy(
                    src_hbm.at[wid, 0], srcb[(k + 1) % 4],
                    isem_s[(k + 1) % 4]).wait()
                pltpu.async_copy(h_hbm.at[srcb[(k + 1) % 4]], rowb[1 - b],
                                 gsem[1 - b])

            @pl.when(c + 3 < NCH)
            def _():
                pltpu.async_copy(src_hbm.at[wid, c + 3], srcb[(k + 3) % 4],
                                 isem_s[(k + 3) % 4])
                pltpu.async_copy(dst_hbm.at[wid, c + 3], dstb[(k + 3) % 4],
                                 isem_d[(k + 3) % 4])

            pltpu.make_async_copy(h_hbm.at[srcb[k]], rowb[b], gsem[b]).wait()

            def _sgrp(g, _):
                w16 = wbuf[b][pl.ds(g * 16, 16)]
                for j in range(16):
                    ws = _splat(w16, j)
                    r = g * 16 + j
                    for f in range(D // 16):
                        sl = pl.ds(f * 16, 16)
                        rowb[b][r, sl] = rowb[b][r, sl] * ws
                return 0
            lax.fori_loop(0, CH // 16, _sgrp, 0)

            pltpu.async_copy(rowb[b], acc_sp.at[dstb[k]], ssem[b], add=True)
        return 0

    lax.fori_loop(0, NCH // 4, _quad, 0)
    pltpu.make_async_copy(rowb[(NCH - 1) % 2],
                          acc_sp.at[dstb[(NCH - 1) % 4]],
                          ssem[(NCH - 1) % 2]).wait()
    plsc.subcore_barrier()

    pltpu.sync_copy(acc_sp.at[pl.ds(sid * RPT, RPT)],
                    outp_hbm.at[cid, pl.ds(sid * RPT, RPT)])
    pltpu.sync_copy(den_sp.at[pl.ds(sid * RPT, RPT)],
                    denp_hbm.at[cid, pl.ds(sid * RPT, RPT)])


def _alpha_epilogue(h, asr_ref, adr_ref, as_ref, ad_ref, c_ref, m_ref, i):
    asv = jnp.sum(h * asr_ref[...], axis=1)
    adv = jnp.sum(h * adr_ref[...], axis=1)
    as_ref[0, 0, :] = asv
    ad_ref[0, 0, :] = adv

    @pl.when(i == 0)
    def _():
        m_ref[0] = jnp.max(asv)
        m_ref[1] = jnp.max(adv)

    @pl.when(i > 0)
    def _():
        m_ref[0] = jnp.maximum(m_ref[0], jnp.max(asv))
        m_ref[1] = jnp.maximum(m_ref[1], jnp.max(adv))

    @pl.when(i == pl.num_programs(0) - 1)
    def _():
        m = m_ref[0] + m_ref[1]
        c = jnp.where(m >= 0.0, m, 0.2 * m)
        c_ref[...] = jnp.full((1, 128), c, jnp.float32)


def _tc_pre_body(x_ref, w_ref, asr_ref, adr_ref,
                 h_ref, as_ref, ad_ref, c_ref, m_ref):
    i = pl.program_id(0)
    h = jnp.dot(x_ref[...], w_ref[...], preferred_element_type=jnp.float32)
    h_ref[...] = h
    _alpha_epilogue(h, asr_ref, adr_ref, as_ref, ad_ref, c_ref, m_ref, i)


def _tc_mid_body(p_ref, d_ref, b_ref, w_ref, asr_ref, adr_ref,
                 h_ref, as_ref, ad_ref, c_ref, m_ref):
    i = pl.program_id(0)
    accs = p_ref[0] + p_ref[1]
    den = d_ref[0, 0, 0, :] + d_ref[1, 0, 0, :]
    hin = accs / (den[:, None] + 1e-16) + b_ref[...]
    hin = jnp.maximum(hin, 0.0)
    h = jnp.dot(hin, w_ref[...], preferred_element_type=jnp.float32)
    h_ref[...] = h
    _alpha_epilogue(h, asr_ref, adr_ref, as_ref, ad_ref, c_ref, m_ref, i)


def _tc_post_body(p_ref, d_ref, b_ref, o_ref):
    accs = p_ref[0] + p_ref[1]
    den = d_ref[0, 0, 0, :] + d_ref[1, 0, 0, :]
    o_ref[...] = accs / (den[:, None] + 1e-16) + b_ref[...]


_vec_specs = [
    pl.BlockSpec((1, D), lambda i: (0, 0)),
    pl.BlockSpec((1, D), lambda i: (0, 0)),
]
_alpha_out_shapes = (
    jax.ShapeDtypeStruct((NP, D), jnp.float32),
    jax.ShapeDtypeStruct((GB, 1, MBLK), jnp.float32),
    jax.ShapeDtypeStruct((GB, 1, MBLK), jnp.float32),
    jax.ShapeDtypeStruct((1, 128), jnp.float32),
)
_alpha_out_specs = [
    pl.BlockSpec((MBLK, D), lambda i: (i, 0)),
    pl.BlockSpec((1, 1, MBLK), lambda i: (i, 0, 0)),
    pl.BlockSpec((1, 1, MBLK), lambda i: (i, 0, 0)),
    pl.BlockSpec((1, 128), lambda i: (0, 0)),
]

_tc_pre = pl.pallas_call(
    _tc_pre_body,
    grid=(GB,),
    in_specs=[
        pl.BlockSpec((MBLK, D), lambda i: (i, 0)),
        pl.BlockSpec((D, D), lambda i: (0, 0)),
    ] + _vec_specs,
    out_specs=_alpha_out_specs,
    out_shape=_alpha_out_shapes,
    scratch_shapes=[pltpu.SMEM((2,), jnp.float32)],
)

_part_specs = [
    pl.BlockSpec((NC, MBLK, D), lambda i: (0, i, 0)),
    pl.BlockSpec((NC, 1, 1, MBLK), lambda i: (0, i, 0, 0)),
    pl.BlockSpec((1, D), lambda i: (0, 0)),
]

_tc_mid = pl.pallas_call(
    _tc_mid_body,
    grid=(GB,),
    in_specs=_part_specs + [
        pl.BlockSpec((D, D), lambda i: (0, 0)),
    ] + _vec_specs,
    out_specs=_alpha_out_specs,
    out_shape=_alpha_out_shapes,
    scratch_shapes=[pltpu.SMEM((2,), jnp.float32)],
)

_tc_post = pl.pallas_call(
    _tc_post_body,
    grid=(GB,),
    in_specs=_part_specs,
    out_specs=pl.BlockSpec((MBLK, D), lambda i: (i, 0)),
    out_shape=jax.ShapeDtypeStruct((NP, D), jnp.float32),
)


@jax.jit
def kernel(x, edge_index, W1, a_src1, a_dst1, b1, W2, a_src2, a_dst2, b2):
    src = edge_index[0].astype(jnp.int32)
    dst = edge_index[1].astype(jnp.int32)

    x_p = jnp.zeros((NP, D), jnp.float32).at[:N].set(x)

    n_pad = EPW_PAD - EPW
    pad_i = (jnp.arange(n_pad, dtype=jnp.int32)[None, :] * 131
             + jnp.arange(NW, dtype=jnp.int32)[:, None] * 257) % N
    src3 = jnp.concatenate([src.reshape(NW, EPW), pad_i], axis=1)
    src3 = src3.reshape(NW, NCH, CH)
    dst3 = jnp.concatenate([dst.reshape(NW, EPW), pad_i], axis=1)
    dst3 = dst3.reshape(NW, NCH, CH)

    def edge_stage(h, as3, ad3, cvec):
        return _sc_edge(h, as3.reshape(NP), ad3.reshape(NP),
                        cvec[0, :16], src3, dst3)

    h1, as1, ad1, c1 = _tc_pre(x_p, W1, a_src1.reshape(1, D),
                               a_dst1.reshape(1, D))
    p1, d1 = edge_stage(h1, as1, ad1, c1)
    h2, as2, ad2, c2 = _tc_mid(p1, d1.reshape(NC, GB, 1, MBLK),
                               b1.reshape(1, D), W2,
                               a_src2.reshape(1, D), a_dst2.reshape(1, D))
    p2, d2 = edge_stage(h2, as2, ad2, c2)
    out = _tc_post(p2, d2.reshape(NC, GB, 1, MBLK), b2.reshape(1, D))
    return out[:N]


# speedup vs baseline: 53.2221x; 1.0749x over previous
import functools

import jax
import jax.numpy as jnp
from jax import lax
from jax.experimental import pallas as pl
from jax.experimental.pallas import tpu as pltpu
from jax.experimental.pallas import tpu_sc as plsc

N = 10000
E = 320000
D = 128
NP = 10240
NC, NS, L = 2, 16, 16
NW = NC * NS
EPW = E // NW
CH = 96
NCH = 108
EPW_PAD = NCH * CH
MBLK = 1024
GB = NP // MBLK
RPT = NP // NS


def _splat(v16, j):
    idx = jnp.full((16, 1), j, jnp.int32)
    dnums = lax.GatherDimensionNumbers(
        offset_dims=(), collapsed_slice_dims=(0,), start_index_map=(0,))
    return lax.gather(v16, idx, dnums, (1,),
                      mode=lax.GatherScatterMode.PROMISE_IN_BOUNDS)


@functools.partial(
    pl.kernel,
    out_type=(
        jax.ShapeDtypeStruct((NC, NP, D), jnp.float32),
        jax.ShapeDtypeStruct((NC, NP), jnp.float32),
    ),
    mesh=plsc.VectorSubcoreMesh(core_axis_name="c", subcore_axis_name="s"),
    compiler_params=pltpu.CompilerParams(needs_layout_passes=False),
    scratch_types=[
        pltpu.VMEM((NP,), jnp.float32),
        pltpu.VMEM((NP,), jnp.float32),
        pltpu.VMEM((16,), jnp.float32),
        pltpu.VMEM((CH,), jnp.int32),
        pltpu.VMEM((CH,), jnp.int32),
        pltpu.VMEM((CH,), jnp.int32),
        pltpu.VMEM((CH,), jnp.int32),
        pltpu.VMEM((CH,), jnp.int32),
        pltpu.VMEM((CH,), jnp.int32),
        pltpu.VMEM((CH,), jnp.int32),
        pltpu.VMEM((CH,), jnp.int32),
        pltpu.VMEM((CH, D), jnp.float32),
        pltpu.VMEM((CH, D), jnp.float32),
        pltpu.VMEM((CH,), jnp.float32),
        pltpu.VMEM((CH,), jnp.float32),
        pltpu.VMEM_SHARED((NP, D), jnp.float32),
        pltpu.VMEM_SHARED((NP,), jnp.float32),
        pltpu.SemaphoreType.DMA,
        pltpu.SemaphoreType.DMA,
        pltpu.SemaphoreType.DMA,
        pltpu.SemaphoreType.DMA,
        pltpu.SemaphoreType.DMA,
        pltpu.SemaphoreType.DMA,
        pltpu.SemaphoreType.DMA,
        pltpu.SemaphoreType.DMA,
        pltpu.SemaphoreType.DMA,
        pltpu.SemaphoreType.DMA,
        pltpu.SemaphoreType.DMA,
        pltpu.SemaphoreType.DMA,
        pltpu.SemaphoreType.DMA,
        pltpu.SemaphoreType.DMA,
    ],
)
def _sc_edge(h_hbm, as_hbm, ad_hbm, c_hbm, src_hbm, dst_hbm,
             outp_hbm, denp_hbm,
             as_v, ad_v, c_v,
             sb0, sb1, sb2, sb3, db0, db1, db2, db3,
             rb0, rb1, wb0, wb1, acc_sp, den_sp,
             gs0, gs1, ss0, ss1,
             is0, is1, is2, is3, id0, id1, id2, id3, ds0, ds1):
    cid = lax.axis_index("c")
    sid = lax.axis_index("s")
    wid = cid * NS + sid
    srcb = (sb0, sb1, sb2, sb3)
    dstb = (db0, db1, db2, db3)
    rowb = (rb0, rb1)
    wbuf = (wb0, wb1)
    gsem = (gs0, gs1)
    ssem = (ss0, ss1)
    isem_s = (is0, is1, is2, is3)
    isem_d = (id0, id1, id2, id3)
    dsem = (ds0, ds1)

    pltpu.sync_copy(as_hbm, as_v)
    pltpu.sync_copy(ad_hbm, ad_v)
    pltpu.sync_copy(c_hbm, c_v)

    def _zr(i, _):
        rb0[i // 8, pl.ds((i % 8) * 16, 16)] = jnp.zeros((16,), jnp.float32)
        return 0
    lax.fori_loop(0, CH * 8, _zr, 0)

    def _zw(i, _):
        wb0[pl.ds(i * 16, 16)] = jnp.zeros((16,), jnp.float32)
        return 0
    lax.fori_loop(0, CH // 16, _zw, 0)

    for t in range(RPT // CH):
        pltpu.sync_copy(rb0, acc_sp.at[pl.ds(sid * RPT + t * CH, CH)])
        pltpu.sync_copy(wb0, den_sp.at[pl.ds(sid * RPT + t * CH, CH)])
    rem = RPT - (RPT // CH) * CH
    if rem:
        base = sid * RPT + (RPT // CH) * CH
        pltpu.sync_copy(rb0.at[pl.ds(0, rem)], acc_sp.at[pl.ds(base, rem)])
        pltpu.sync_copy(wb0.at[pl.ds(0, rem)], den_sp.at[pl.ds(base, rem)])
    plsc.subcore_barrier()

    c_off = c_v[...]

    for t in range(3):
        pltpu.async_copy(src_hbm.at[wid, t], srcb[t], isem_s[t])
        pltpu.async_copy(dst_hbm.at[wid, t], dstb[t], isem_d[t])
    pltpu.make_async_copy(src_hbm.at[wid, 0], srcb[0], isem_s[0]).wait()
    pltpu.async_copy(h_hbm.at[srcb[0]], rb0, gsem[0])

    def _quad(i, _):
        for k in range(4):
            c = i * 4 + k
            b = k % 2
            pltpu.make_async_copy(
                dst_hbm.at[wid, 0], dstb[k], isem_d[k]).wait()

            def _wgrp(g, _):
                s = srcb[k][pl.ds(g * 16, 16)]
                d = dstb[k][pl.ds(g * 16, 16)]
                e = plsc.load_gather(as_v, [s]) + plsc.load_gather(ad_v, [d])
                e = jnp.where(e >= 0.0, e, 0.2 * e)
                w = jnp.exp(e - c_off)
                pos = c * CH + g * 16 + lax.iota(jnp.int32, 16)
                w = jnp.where(pos < EPW, w, 0.0)
                wbuf[b][pl.ds(g * 16, 16)] = w
                return 0
            lax.fori_loop(0, CH // 16, _wgrp, 0)

            pltpu.async_copy(wbuf[b], den_sp.at[dstb[k]], dsem[b], add=True)

            @pl.when(c >= 1)
            def _():
                pltpu.make_async_copy(
                    rowb[1 - b], acc_sp.at[dstb[(k - 1) % 4]],
                    ssem[1 - b]).wait()
                pltpu.make_async_copy(
                    wbuf[1 - b], den_sp.at[dstb[(k - 1) % 4]],
                    dsem[1 - b]).wait()

            @pl.when(c + 1 < NCH)
            def _():
                pltpu.make_async_copy(
                    src_hbm.at[wid, 0], srcb[(k + 1) % 4],
                    isem_s[(k + 1) % 4]).wait()
                pltpu.async_copy(h_hbm.at[srcb[(k + 1) % 4]], rowb[1 - b],
                                 gsem[1 - b])

            @pl.when(c + 3 < NCH)
            def _():
                pltpu.async_copy(src_hbm.at[wid, c + 3], srcb[(k + 3) % 4],
                                 isem_s[(k + 3) % 4])
                pltpu.async_copy(dst_hbm.at[wid, c + 3], dstb[(k + 3) % 4],
                                 isem_d[(k + 3) % 4])

            pltpu.make_async_copy(h_hbm.at[srcb[k]], rowb[b], gsem[b]).wait()

            def _sgrp(g, _):
                w16 = wbuf[b][pl.ds(g * 16, 16)]
                for j in range(16):
                    ws = _splat(w16, j)
                    r = g * 16 + j
                    for f in range(D // 16):
                        sl = pl.ds(f * 16, 16)
                        rowb[b][r, sl] = rowb[b][r, sl] * ws
                return 0
            lax.fori_loop(0, CH // 16, _sgrp, 0)

            pltpu.async_copy(rowb[b], acc_sp.at[dstb[k]], ssem[b], add=True)
        return 0

    lax.fori_loop(0, NCH // 4, _quad, 0)
    pltpu.make_async_copy(rowb[(NCH - 1) % 2],
                          acc_sp.at[dstb[(NCH - 1) % 4]],
                          ssem[(NCH - 1) % 2]).wait()
    pltpu.make_async_copy(wbuf[(NCH - 1) % 2],
                          den_sp.at[dstb[(NCH - 1) % 4]],
                          dsem[(NCH - 1) % 2]).wait()
    plsc.subcore_barrier()

    pltpu.sync_copy(acc_sp.at[pl.ds(sid * RPT, RPT)],
                    outp_hbm.at[cid, pl.ds(sid * RPT, RPT)])
    pltpu.sync_copy(den_sp.at[pl.ds(sid * RPT, RPT)],
                    denp_hbm.at[cid, pl.ds(sid * RPT, RPT)])


def _alpha_epilogue(h, asr_ref, adr_ref, as_ref, ad_ref, c_ref, m_ref, i):
    asv = jnp.sum(h * asr_ref[...], axis=1)
    adv = jnp.sum(h * adr_ref[...], axis=1)
    as_ref[0, 0, :] = asv
    ad_ref[0, 0, :] = adv

    @pl.when(i == 0)
    def _():
        m_ref[0] = jnp.max(asv)
        m_ref[1] = jnp.max(adv)

    @pl.when(i > 0)
    def _():
        m_ref[0] = jnp.maximum(m_ref[0], jnp.max(asv))
        m_ref[1] = jnp.maximum(m_ref[1], jnp.max(adv))

    @pl.when(i == pl.num_programs(0) - 1)
    def _():
        m = m_ref[0] + m_ref[1]
        c = jnp.where(m >= 0.0, m, 0.2 * m)
        c_ref[...] = jnp.full((1, 128), c, jnp.float32)


def _tc_pre_body(x_ref, w_ref, asr_ref, adr_ref,
                 h_ref, as_ref, ad_ref, c_ref, m_ref):
    i = pl.program_id(0)
    h = jnp.dot(x_ref[...], w_ref[...], preferred_element_type=jnp.float32)
    h_ref[...] = h
    _alpha_epilogue(h, asr_ref, adr_ref, as_ref, ad_ref, c_ref, m_ref, i)


def _tc_mid_body(p_ref, d_ref, b_ref, w_ref, asr_ref, adr_ref,
                 h_ref, as_ref, ad_ref, c_ref, m_ref):
    i = pl.program_id(0)
    accs = p_ref[0] + p_ref[1]
    den = d_ref[0, 0, 0, :] + d_ref[1, 0, 0, :]
    hin = accs / (den[:, None] + 1e-16) + b_ref[...]
    hin = jnp.maximum(hin, 0.0)
    h = jnp.dot(hin, w_ref[...], preferred_element_type=jnp.float32)
    h_ref[...] = h
    _alpha_epilogue(h, asr_ref, adr_ref, as_ref, ad_ref, c_ref, m_ref, i)


def _tc_post_body(p_ref, d_ref, b_ref, o_ref):
    accs = p_ref[0] + p_ref[1]
    den = d_ref[0, 0, 0, :] + d_ref[1, 0, 0, :]
    o_ref[...] = accs / (den[:, None] + 1e-16) + b_ref[...]


_vec_specs = [
    pl.BlockSpec((1, D), lambda i: (0, 0)),
    pl.BlockSpec((1, D), lambda i: (0, 0)),
]
_alpha_out_shapes = (
    jax.ShapeDtypeStruct((NP, D), jnp.float32),
    jax.ShapeDtypeStruct((GB, 1, MBLK), jnp.float32),
    jax.ShapeDtypeStruct((GB, 1, MBLK), jnp.float32),
    jax.ShapeDtypeStruct((1, 128), jnp.float32),
)
_alpha_out_specs = [
    pl.BlockSpec((MBLK, D), lambda i: (i, 0)),
    pl.BlockSpec((1, 1, MBLK), lambda i: (i, 0, 0)),
    pl.BlockSpec((1, 1, MBLK), lambda i: (i, 0, 0)),
    pl.BlockSpec((1, 128), lambda i: (0, 0)),
]

_tc_pre = pl.pallas_call(
    _tc_pre_body,
    grid=(GB,),
    in_specs=[
        pl.BlockSpec((MBLK, D), lambda i: (i, 0)),
        pl.BlockSpec((D, D), lambda i: (0, 0)),
    ] + _vec_specs,
    out_specs=_alpha_out_specs,
    out_shape=_alpha_out_shapes,
    scratch_shapes=[pltpu.SMEM((2,), jnp.float32)],
)

_part_specs = [
    pl.BlockSpec((NC, MBLK, D), lambda i: (0, i, 0)),
    pl.BlockSpec((NC, 1, 1, MBLK), lambda i: (0, i, 0, 0)),
    pl.BlockSpec((1, D), lambda i: (0, 0)),
]

_tc_mid = pl.pallas_call(
    _tc_mid_body,
    grid=(GB,),
    in_specs=_part_specs + [
        pl.BlockSpec((D, D), lambda i: (0, 0)),
    ] + _vec_specs,
    out_specs=_alpha_out_specs,
    out_shape=_alpha_out_shapes,
    scratch_shapes=[pltpu.SMEM((2,), jnp.float32)],
)

_tc_post = pl.pallas_call(
    _tc_post_body,
    grid=(GB,),
    in_specs=_part_specs,
    out_specs=pl.BlockSpec((MBLK, D), lambda i: (i, 0)),
    out_shape=jax.ShapeDtypeStruct((NP, D), jnp.float32),
)


@jax.jit
def kernel(x, edge_index, W1, a_src1, a_dst1, b1, W2, a_src2, a_dst2, b2):
    src = edge_index[0].astype(jnp.int32)
    dst = edge_index[1].astype(jnp.int32)

    x_p = jnp.zeros((NP, D), jnp.float32).at[:N].set(x)

    n_pad = EPW_PAD - EPW
    pad_i = (jnp.arange(n_pad, dtype=jnp.int32)[None, :] * 131
             + jnp.arange(NW, dtype=jnp.int32)[:, None] * 257) % N
    src3 = jnp.concatenate([src.reshape(NW, EPW), pad_i], axis=1)
    src3 = src3.reshape(NW, NCH, CH)
    dst3 = jnp.concatenate([dst.reshape(NW, EPW), pad_i], axis=1)
    dst3 = dst3.reshape(NW, NCH, CH)

    def edge_stage(h, as3, ad3, cvec):
        return _sc_edge(h, as3.reshape(NP), ad3.reshape(NP),
                        cvec[0, :16], src3, dst3)

    h1, as1, ad1, c1 = _tc_pre(x_p, W1, a_src1.reshape(1, D),
                               a_dst1.reshape(1, D))
    p1, d1 = edge_stage(h1, as1, ad1, c1)
    h2, as2, ad2, c2 = _tc_mid(p1, d1.reshape(NC, GB, 1, MBLK),
                               b1.reshape(1, D), W2,
                               a_src2.reshape(1, D), a_dst2.reshape(1, D))
    p2, d2 = edge_stage(h2, as2, ad2, c2)
    out = _tc_post(p2, d2.reshape(NC, GB, 1, MBLK), b2.reshape(1, D))
    return out[:N]
